# baseline (speedup 1.0000x reference)
import functools

import jax
import jax.numpy as jnp
from jax import lax
from jax.experimental import pallas as pl
from jax.experimental.pallas import tpu as pltpu
from jax.experimental.pallas import tpu_sc as plsc

B = 16384
D = 32
NC = 2
NS = 16
NW = NC * NS
BPW = B // NW
CH = 128
NCHUNK = BPW // CH
L = 16

_mesh = plsc.VectorSubcoreMesh(
    core_axis_name="c", subcore_axis_name="s", num_cores=NC, num_subcores=NS
)


@functools.partial(
    pl.kernel,
    out_type=(
        jax.ShapeDtypeStruct((B,), jnp.float32),
        jax.ShapeDtypeStruct((B, D), jnp.float32),
        jax.ShapeDtypeStruct((B, D), jnp.float32),
    ),
    mesh=_mesh,
    scratch_types=[
        pltpu.VMEM((BPW,), jnp.int32),
        pltpu.VMEM((BPW,), jnp.int32),
        pltpu.VMEM((BPW, D), jnp.float32),
        pltpu.VMEM((BPW, D), jnp.float32),
        pltpu.VMEM((BPW, D), jnp.float32),
        pltpu.VMEM((BPW, D), jnp.float32),
        pltpu.VMEM((D, L), jnp.float32),
        pltpu.VMEM((BPW,), jnp.float32),
        pltpu.SemaphoreType.DMA,
        pltpu.SemaphoreType.DMA,
    ],
    compiler_params=pltpu.CompilerParams(
        needs_layout_passes=False, use_tc_tiling_on_sc=False),
)
def _sc_gather(uidx_hbm, iidx_hbm, gu_hbm, gi_hbm, mu_hbm, mi_hbm, wo_hbm,
               dot_hbm, mlpu_hbm, mlpi_hbm,
               uidx_v, iidx_v, gu_v, gi_v, mu_v, mi_v, wo_v, dot_v,
               sem_g, sem_m):
    wid = lax.axis_index("s") * NC + lax.axis_index("c")
    base = wid * BPW
    pltpu.sync_copy(uidx_hbm.at[pl.ds(base, BPW)], uidx_v)
    pltpu.sync_copy(iidx_hbm.at[pl.ds(base, BPW)], iidx_v)
    pltpu.sync_copy(wo_hbm, wo_v)

    gmf_copies, mlp_copies = [], []
    for j in range(NCHUNK):
        sl = pl.ds(j * CH, CH)
        gmf_copies.append(
            pltpu.async_copy(gu_hbm.at[uidx_v.at[sl]], gu_v.at[sl], sem_g))
        gmf_copies.append(
            pltpu.async_copy(gi_hbm.at[iidx_v.at[sl]], gi_v.at[sl], sem_g))
        mlp_copies.append(
            pltpu.async_copy(mu_hbm.at[uidx_v.at[sl]], mu_v.at[sl], sem_m))
        mlp_copies.append(
            pltpu.async_copy(mi_hbm.at[iidx_v.at[sl]], mi_v.at[sl], sem_m))
    for c in gmf_copies:
        c.wait()

    def group_body(g, carry):
        rows = g * L + lax.iota(jnp.int32, L)
        acc = jnp.zeros((L,), jnp.float32)
        for c in range(D):
            colv = jnp.full((L,), c, jnp.int32)
            u = plsc.load_gather(gu_v, [rows, colv])
            it = plsc.load_gather(gi_v, [rows, colv])
            acc = acc + u * it * wo_v[c]
        dot_v[pl.ds(g * L, L)] = acc
        return carry

    lax.fori_loop(0, BPW // L, group_body, 0)

    for c in mlp_copies:
        c.wait()
    pltpu.sync_copy(dot_v, dot_hbm.at[pl.ds(base, BPW)])
    pltpu.sync_copy(mu_v, mlpu_hbm.at[pl.ds(base, BPW)])
    pltpu.sync_copy(mi_v, mlpi_hbm.at[pl.ds(base, BPW)])


BM = 2048


def _tc_body(dot_ref, mu_ref, mi_ref, w1a_ref, w1b_ref, b1_ref, w2_ref,
             b2_ref, w3_ref, b3_ref, wom_ref, bo_ref, out_ref):
    f32 = jnp.float32
    h = jnp.dot(mu_ref[...], w1a_ref[...], preferred_element_type=f32)
    h = h + jnp.dot(mi_ref[...], w1b_ref[...], preferred_element_type=f32)
    h = jnp.maximum(h + b1_ref[...], 0.0)
    h = jnp.maximum(
        jnp.dot(h, w2_ref[...], preferred_element_type=f32) + b2_ref[...], 0.0)
    h = jnp.maximum(
        jnp.dot(h, w3_ref[...], preferred_element_type=f32) + b3_ref[...], 0.0)
    logit = (jnp.dot(h, wom_ref[...], preferred_element_type=f32)
             + dot_ref[...] + bo_ref[...])
    out_ref[...] = 1.0 / (1.0 + jnp.exp(-logit))


_tc_mlp = pl.pallas_call(
    _tc_body,
    grid=(B // BM,),
    in_specs=[
        pl.BlockSpec((BM, 1), lambda i: (i, 0)),
        pl.BlockSpec((BM, D), lambda i: (i, 0)),
        pl.BlockSpec((BM, D), lambda i: (i, 0)),
        pl.BlockSpec((D, D), lambda i: (0, 0)),
        pl.BlockSpec((D, D), lambda i: (0, 0)),
        pl.BlockSpec((1, D), lambda i: (0, 0)),
        pl.BlockSpec((D, 16), lambda i: (0, 0)),
        pl.BlockSpec((1, 16), lambda i: (0, 0)),
        pl.BlockSpec((16, 8), lambda i: (0, 0)),
        pl.BlockSpec((1, 8), lambda i: (0, 0)),
        pl.BlockSpec((8, 1), lambda i: (0, 0)),
        pl.BlockSpec((1, 1), lambda i: (0, 0)),
    ],
    out_specs=pl.BlockSpec((BM, 1), lambda i: (i, 0)),
    out_shape=jax.ShapeDtypeStruct((B, 1), jnp.float32),
)


def kernel(user_indices, item_indices, gmf_user_table, gmf_item_table,
           mlp_user_table, mlp_item_table, W1, b1, W2, b2, W3, b3, Wo, bo):
    wo_gmf = jnp.broadcast_to(Wo[:D], (D, L))
    gmf_dot, mlp_u, mlp_i = _sc_gather(
        user_indices, item_indices, gmf_user_table, gmf_item_table,
        mlp_user_table, mlp_item_table, wo_gmf)
    out2d = _tc_mlp(
        gmf_dot.reshape(B, 1), mlp_u, mlp_i,
        W1[:D], W1[D:], b1.reshape(1, -1),
        W2, b2.reshape(1, -1), W3, b3.reshape(1, -1),
        Wo[D:], bo.reshape(1, 1))
    return out2d.reshape(B)
